# baseline (speedup 1.0000x reference)
import dataclasses
import functools

import jax
import jax.numpy as jnp
from jax import lax
from jax.experimental import pallas as pl
from jax.experimental.pallas import tpu as pltpu
from jax.experimental.pallas import tpu_sc as plsc

N = 10000
E = 320000
D = 128
N_PAD = 10240
TILES = 32
W = 128
NW = 80
E_PAD = TILES * NW * W
RPS = N_PAD // 16
DEG_W = 128
NBUF = 2
PH = 2
NWP = NW // PH


def _mesh():
    return plsc.VectorSubcoreMesh(core_axis_name="c", subcore_axis_name="s")


DEG_R = N_PAD // D


def _sc_deg(dst_r, rowidx, zrows):
    @functools.partial(
        pl.kernel,
        out_type=jax.ShapeDtypeStruct((2, DEG_R, D), jnp.float32),
        mesh=_mesh(),
        compiler_params=_compiler_params(),
        scratch_types=[
            pltpu.VMEM((NW, W), jnp.int32),
            pltpu.VMEM((1, DEG_R), jnp.int32),
            pltpu.VMEM((DEG_R, D), jnp.float32),
            pltpu.VMEM_SHARED((DEG_R, D), jnp.float32),
        ],
    )
    def k(dst_hbm, rowidx_hbm, z_hbm, out_hbm, dst_v, rix_v, hist_v, deg_sh):
        c = lax.axis_index("c")
        s = lax.axis_index("s")
        wid = c * 16 + s

        @pl.when(s == 0)
        def _():
            pltpu.sync_copy(z_hbm.at[pl.ds(0, DEG_R)], deg_sh)

        pltpu.sync_copy(dst_hbm.at[wid], dst_v)
        pltpu.sync_copy(rowidx_hbm, rix_v)

        @pl.loop(0, DEG_R)
        def _(r):
            @pl.loop(0, D, step=16)
            def _(j):
                hist_v[r, pl.ds(j, 16)] = jnp.zeros((16,), jnp.float32)

        ones16 = jnp.ones((16,), jnp.float32)
        m127 = jnp.full((16,), 127, jnp.int32)
        sh7 = jnp.full((16,), 7, jnp.int32)

        @pl.loop(0, NW)
        def _(w):
            @pl.loop(0, W, step=16)
            def _(j):
                ix = dst_v[w, pl.ds(j, 16)]
                ir = lax.shift_right_logical(ix, sh7)
                ic = lax.bitwise_and(ix, m127)
                plsc.addupdate_scatter(hist_v, [ir, ic], ones16)

        plsc.subcore_barrier()
        pltpu.sync_copy(hist_v, deg_sh.at[rix_v.at[0]], add=True)
        plsc.subcore_barrier()

        @pl.when(s < 5)
        def _():
            pltpu.sync_copy(
                deg_sh.at[pl.ds(s * 16, 16)], out_hbm.at[c, pl.ds(s * 16, 16)]
            )

    return k(dst_r, rowidx, zrows)


def _compiler_params():
    cp = pltpu.CompilerParams()
    if "needs_layout_passes" in pltpu.CompilerParams.__dataclass_fields__:
        cp = dataclasses.replace(cp, needs_layout_passes=False)
    return cp


def _sc_pass(hb, src_r, dst_r, zrows):
    @functools.partial(
        pl.kernel,
        out_type=jax.ShapeDtypeStruct((2, N_PAD, D), jnp.float32),
        mesh=_mesh(),
        compiler_params=_compiler_params(),
        scratch_types=[
            pltpu.VMEM((NWP, W), jnp.int32),
            pltpu.VMEM((NWP, W), jnp.int32),
            pltpu.VMEM_SHARED((N_PAD, D), jnp.float32),
        ]
        + [pltpu.VMEM((W, D), jnp.float32) for _ in range(NBUF)]
        + [pltpu.SemaphoreType.DMA for _ in range(NBUF)],
    )
    def k(h_hbm, src_hbm, dst_hbm, z_hbm, out_hbm, src_v, dst_v,
          acc_sh, *bufs_sems):
        rows = bufs_sems[:NBUF]
        sems = bufs_sems[NBUF : 2 * NBUF]
        c = lax.axis_index("c")
        s = lax.axis_index("s")
        wid = c * 16 + s

        pltpu.sync_copy(z_hbm, acc_sh.at[pl.ds(s * RPS, RPS)])
        plsc.subcore_barrier()

        for p in range(PH):
            pltpu.sync_copy(src_hbm.at[wid, pl.ds(p * NWP, NWP)], src_v)
            pltpu.sync_copy(dst_hbm.at[wid, pl.ds(p * NWP, NWP)], dst_v)

            for b in range(NBUF):
                pltpu.async_copy(h_hbm.at[src_v.at[b]], rows[b], sems[b])

            @pl.loop(0, NWP, step=NBUF)
            def _(w):
                for b in range(NBUF):
                    pltpu.make_async_copy(
                        h_hbm.at[src_v.at[w + b]], rows[b], sems[b]
                    ).wait()
                    pltpu.sync_copy(rows[b], acc_sh.at[dst_v.at[w + b]], add=True)
                    nxt = jnp.minimum(w + b + NBUF, NWP - 1)
                    pltpu.async_copy(h_hbm.at[src_v.at[nxt]], rows[b], sems[b])

            for b in range(NBUF):
                pltpu.make_async_copy(
                    h_hbm.at[src_v.at[NWP - 1]], rows[b], sems[b]
                ).wait()

        plsc.subcore_barrier()
        pltpu.sync_copy(
            acc_sh.at[pl.ds(s * RPS, RPS)], out_hbm.at[c, pl.ds(s * RPS, RPS)]
        )

    return k(hb, src_r, dst_r, zrows)


def _dinv_from(deg0, deg1):
    d = (deg0 + deg1).reshape(N_PAD)[0:N, None] + 1.0
    return jax.lax.rsqrt(d)


def _tc_mm_scale(x, Wmat, deg2):
    def body(x_ref, w_ref, deg_ref, o_ref):
        dinv = _dinv_from(deg_ref[0], deg_ref[1])
        h = jax.lax.dot_general(
            x_ref[...], w_ref[...], (((1,), (1,)), ((), ())),
            preferred_element_type=jnp.float32,
        )
        o_ref[...] = h * dinv

    return pl.pallas_call(
        body, out_shape=jax.ShapeDtypeStruct((N, D), jnp.float32)
    )(x, Wmat, deg2)


def _tc_mid(acc, h1s, deg2, b, Wmat):
    def body(acc_ref, h1s_ref, deg_ref, b_ref, w_ref, o_ref):
        dinv = _dinv_from(deg_ref[0], deg_ref[1])
        ssum = acc_ref[0, 0:N, :] + acc_ref[1, 0:N, :] + h1s_ref[...]
        z = jnp.maximum(ssum * dinv + b_ref[...], 0.0)
        h2 = jax.lax.dot_general(
            z, w_ref[...], (((1,), (1,)), ((), ())),
            preferred_element_type=jnp.float32,
        )
        o_ref[...] = h2 * dinv

    return pl.pallas_call(
        body, out_shape=jax.ShapeDtypeStruct((N, D), jnp.float32)
    )(acc, h1s, deg2, b.reshape(1, D), Wmat)


def _tc_out(acc, h2s, deg2, b):
    def body(acc_ref, h2s_ref, deg_ref, b_ref, o_ref):
        dinv = _dinv_from(deg_ref[0], deg_ref[1])
        ssum = acc_ref[0, 0:N, :] + acc_ref[1, 0:N, :] + h2s_ref[...]
        o_ref[...] = ssum * dinv + b_ref[...]

    return pl.pallas_call(
        body, out_shape=jax.ShapeDtypeStruct((N, D), jnp.float32)
    )(acc, h2s, deg2, b.reshape(1, D))


def kernel(x, edge_index, W1, b1, W2, b2):
    src = edge_index[0].astype(jnp.int32)
    dst = edge_index[1].astype(jnp.int32)
    npad = E_PAD - E
    src_pad = jnp.concatenate([src, jnp.zeros((npad,), jnp.int32)])
    dst_pad = jnp.concatenate([dst, jnp.full((npad,), N, jnp.int32)])
    src_r = src_pad.reshape(TILES, NW, W)
    dst_r = dst_pad.reshape(TILES, NW, W)
    rowidx = jnp.arange(N_PAD // D, dtype=jnp.int32).reshape(1, N_PAD // D)
    zrows = jnp.zeros((RPS, D), jnp.float32)

    deg2 = _sc_deg(dst_r, rowidx, zrows)
    h1s = _tc_mm_scale(x, W1, deg2)
    acc1 = _sc_pass(h1s, src_r, dst_r, zrows)
    h2s = _tc_mid(acc1, h1s, deg2, b1, W2)
    acc2 = _sc_pass(h2s, src_r, dst_r, zrows)
    return _tc_out(acc2, h2s, deg2, b2)


# speedup vs baseline: 1.3228x; 1.3228x over previous
import dataclasses
import functools

import jax
import jax.numpy as jnp
from jax import lax
from jax.experimental import pallas as pl
from jax.experimental.pallas import tpu as pltpu
from jax.experimental.pallas import tpu_sc as plsc

N = 10000
E = 320000
D = 128
N_PAD = 10240
TILES = 32
W = 128
NW = 80
E_PAD = TILES * NW * W
RPS = N_PAD // 16
DEG_W = 128
NBUF = 2
PH = 2
NWP = NW // PH


def _mesh():
    return plsc.VectorSubcoreMesh(core_axis_name="c", subcore_axis_name="s")


DEG_R = N_PAD // D


def _sc_deg(dst_r, rowidx, zrows):
    @functools.partial(
        pl.kernel,
        out_type=jax.ShapeDtypeStruct((2, DEG_R, D), jnp.float32),
        mesh=_mesh(),
        compiler_params=_compiler_params(),
        scratch_types=[
            pltpu.VMEM((NW, W), jnp.int32),
            pltpu.VMEM((1, DEG_R), jnp.int32),
            pltpu.VMEM((DEG_R, D), jnp.float32),
            pltpu.VMEM_SHARED((DEG_R, D), jnp.float32),
        ],
    )
    def k(dst_hbm, rowidx_hbm, z_hbm, out_hbm, dst_v, rix_v, hist_v, deg_sh):
        c = lax.axis_index("c")
        s = lax.axis_index("s")
        wid = c * 16 + s

        @pl.when(s == 0)
        def _():
            pltpu.sync_copy(z_hbm.at[pl.ds(0, DEG_R)], deg_sh)

        pltpu.sync_copy(dst_hbm.at[wid], dst_v)
        pltpu.sync_copy(rowidx_hbm, rix_v)

        @pl.loop(0, DEG_R)
        def _(r):
            @pl.loop(0, D, step=16)
            def _(j):
                hist_v[r, pl.ds(j, 16)] = jnp.zeros((16,), jnp.float32)

        ones16 = jnp.ones((16,), jnp.float32)
        m127 = jnp.full((16,), 127, jnp.int32)
        sh7 = jnp.full((16,), 7, jnp.int32)

        @pl.loop(0, NW)
        def _(w):
            @pl.loop(0, W, step=16)
            def _(j):
                ix = dst_v[w, pl.ds(j, 16)]
                ir = lax.shift_right_logical(ix, sh7)
                ic = lax.bitwise_and(ix, m127)
                plsc.addupdate_scatter(hist_v, [ir, ic], ones16)

        plsc.subcore_barrier()
        pltpu.sync_copy(hist_v, deg_sh.at[rix_v.at[0]], add=True)
        plsc.subcore_barrier()

        @pl.when(s < 5)
        def _():
            pltpu.sync_copy(
                deg_sh.at[pl.ds(s * 16, 16)], out_hbm.at[c, pl.ds(s * 16, 16)]
            )

    return k(dst_r, rowidx, zrows)


def _compiler_params():
    cp = pltpu.CompilerParams()
    if "needs_layout_passes" in pltpu.CompilerParams.__dataclass_fields__:
        cp = dataclasses.replace(cp, needs_layout_passes=False)
    return cp


def _sc_pass(hb, src_r, dst_r, zrows):
    @functools.partial(
        pl.kernel,
        out_type=jax.ShapeDtypeStruct((2, N_PAD, D), jnp.float32),
        mesh=_mesh(),
        compiler_params=_compiler_params(),
        scratch_types=[
            pltpu.VMEM((NWP, W), jnp.int32),
            pltpu.VMEM((NWP, W), jnp.int32),
            pltpu.VMEM_SHARED((N_PAD, D), jnp.float32),
        ]
        + [pltpu.VMEM((W, D), jnp.float32) for _ in range(NBUF)]
        + [pltpu.SemaphoreType.DMA for _ in range(NBUF)],
    )
    def k(h_hbm, src_hbm, dst_hbm, z_hbm, out_hbm, src_v, dst_v,
          acc_sh, *bufs_sems):
        rows = bufs_sems[:NBUF]
        sems = bufs_sems[NBUF : 2 * NBUF]
        c = lax.axis_index("c")
        s = lax.axis_index("s")
        wid = c * 16 + s

        pltpu.sync_copy(z_hbm, acc_sh.at[pl.ds(s * RPS, RPS)])
        plsc.subcore_barrier()

        for p in range(PH):
            pltpu.sync_copy(src_hbm.at[wid, pl.ds(p * NWP, NWP)], src_v)
            pltpu.sync_copy(dst_hbm.at[wid, pl.ds(p * NWP, NWP)], dst_v)

            for b in range(NBUF):
                pltpu.async_copy(h_hbm.at[src_v.at[b]], rows[b], sems[b])

            @pl.loop(0, NWP, step=NBUF)
            def _(w):
                for b in range(NBUF):
                    pltpu.make_async_copy(
                        h_hbm.at[src_v.at[w + b]], rows[b], sems[b]
                    ).wait()
                    pltpu.sync_copy(rows[b], acc_sh.at[dst_v.at[w + b]], add=True)
                    nxt = jnp.minimum(w + b + NBUF, NWP - 1)
                    pltpu.async_copy(h_hbm.at[src_v.at[nxt]], rows[b], sems[b])

            for b in range(NBUF):
                pltpu.make_async_copy(
                    h_hbm.at[src_v.at[NWP - 1]], rows[b], sems[b]
                ).wait()

        plsc.subcore_barrier()
        pltpu.sync_copy(
            acc_sh.at[pl.ds(s * RPS, RPS)], out_hbm.at[c, pl.ds(s * RPS, RPS)]
        )

    return k(hb, src_r, dst_r, zrows)


def _tc_mm(x, Wmat):
    def body(x_ref, w_ref, o_ref):
        o_ref[...] = jax.lax.dot_general(
            x_ref[...], w_ref[...], (((1,), (1,)), ((), ())),
            preferred_element_type=jnp.float32,
        )

    return pl.pallas_call(
        body, out_shape=jax.ShapeDtypeStruct((N, D), jnp.float32)
    )(x, Wmat)


def _dinv_from(deg0, deg1):
    d = (deg0 + deg1).reshape(N_PAD)[0:N, None] + 1.0
    return jax.lax.rsqrt(d)


def _tc_scale(h, deg2):
    def body(h_ref, deg_ref, o_ref):
        dinv = _dinv_from(deg_ref[0], deg_ref[1])
        o_ref[...] = h_ref[...] * dinv

    return pl.pallas_call(
        body, out_shape=jax.ShapeDtypeStruct((N, D), jnp.float32)
    )(h, deg2)


def _tc_mid(acc, h1, deg2, b, Wmat):
    def body(acc_ref, h1_ref, deg_ref, b_ref, w_ref, o_ref):
        dinv = _dinv_from(deg_ref[0], deg_ref[1])
        ssum = acc_ref[0, 0:N, :] + acc_ref[1, 0:N, :] + h1_ref[...] * dinv
        z = jnp.maximum(ssum * dinv + b_ref[...], 0.0)
        h2 = jax.lax.dot_general(
            z, w_ref[...], (((1,), (1,)), ((), ())),
            preferred_element_type=jnp.float32,
        )
        o_ref[...] = h2 * dinv

    return pl.pallas_call(
        body, out_shape=jax.ShapeDtypeStruct((N, D), jnp.float32)
    )(acc, h1, deg2, b.reshape(1, D), Wmat)


def _tc_out(acc, h2s, deg2, b):
    def body(acc_ref, h2s_ref, deg_ref, b_ref, o_ref):
        dinv = _dinv_from(deg_ref[0], deg_ref[1])
        ssum = acc_ref[0, 0:N, :] + acc_ref[1, 0:N, :] + h2s_ref[...]
        o_ref[...] = ssum * dinv + b_ref[...]

    return pl.pallas_call(
        body, out_shape=jax.ShapeDtypeStruct((N, D), jnp.float32)
    )(acc, h2s, deg2, b.reshape(1, D))


def kernel(x, edge_index, W1, b1, W2, b2):
    src = edge_index[0].astype(jnp.int32)
    dst = edge_index[1].astype(jnp.int32)
    npad = E_PAD - E
    src_pad = jnp.concatenate([src, jnp.zeros((npad,), jnp.int32)])
    dst_pad = jnp.concatenate([dst, jnp.full((npad,), N, jnp.int32)])
    src_r = src_pad.reshape(TILES, NW, W)
    dst_r = dst_pad.reshape(TILES, NW, W)
    rowidx = jnp.arange(N_PAD // D, dtype=jnp.int32).reshape(1, N_PAD // D)
    zrows = jnp.zeros((RPS, D), jnp.float32)

    deg2 = _sc_deg(dst_r, rowidx, zrows)
    h1 = _tc_mm(x, W1)
    h1s = _tc_scale(h1, deg2)
    acc1 = _sc_pass(h1s, src_r, dst_r, zrows)
    h2s = _tc_mid(acc1, h1, deg2, b1, W2)
    acc2 = _sc_pass(h2s, src_r, dst_r, zrows)
    return _tc_out(acc2, h2s, deg2, b2)
